# baseline (speedup 1.0000x reference)
import jax
import jax.numpy as jnp
from jax import lax
from jax.experimental import pallas as pl
from jax.experimental.pallas import tpu as pltpu
import jax.experimental.pallas.tpu_sc as plsc
from jax.experimental import layout as jex_layout

B = 16
SEQ = 77
N_CTX = 16
N_VERB = 64
CTX_DIM = 512

PAD_SEQ = 96
V_PER_W = N_VERB // 2
LANES = 16
G_ROWS = 80
SLAB = SEQ * CTX_DIM
CTXW = N_CTX * CTX_DIM


def _sc_body(tok_hbm, nn_hbm, pre_hbm, table_hbm, ctxf_hbm,
             out_hbm, ct_hbm,
             tok_v, nn_v, pre_v, idx_v, ct_v, rows_v, slab_a, slab_b,
             sem_a, sem_b, gsem):
    c = lax.axis_index("c")
    s = lax.axis_index("s")
    wid = s * 2 + c
    b = wid // 2
    h = wid % 2
    v0 = h * V_PER_W

    pltpu.sync_copy(tok_hbm.at[pl.ds(b * PAD_SEQ, PAD_SEQ)], tok_v)
    pltpu.sync_copy(nn_hbm, nn_v)
    pltpu.sync_copy(pre_hbm, pre_v)

    bvec = jnp.full((LANES,), b, jnp.int32)
    n1 = plsc.load_gather(nn_v, [bvec])
    n = jnp.max(n1)
    iota = lax.iota(jnp.int32, LANES)

    for ci in range(G_ROWS // LANES):
        j = LANES * ci + iota
        in_ctx = (j > n1) & (j <= n1 + N_CTX)
        tidx = jnp.where(j <= n1, j, j - N_CTX)
        tidx = jnp.clip(tidx, 0, SEQ - 1)
        tok = plsc.load_gather(tok_v, [tidx])
        cidx = jnp.clip(j - 1 - n1, 0, N_CTX - 1)
        pre = plsc.load_gather(pre_v, [cidx])
        ct_v[pl.ds(LANES * ci, LANES)] = jnp.where(in_ctx, pre, tok)
        idx_v[pl.ds(LANES * ci, LANES)] = tok

    @pl.when(h == 0)
    def _():
        pltpu.sync_copy(ct_v, ct_hbm.at[pl.ds(b * PAD_SEQ, G_ROWS)])

    pltpu.async_copy(table_hbm.at[idx_v], rows_v, gsem)
    pltpu.make_async_copy(table_hbm.at[idx_v], rows_v, gsem).wait()

    def _flatten_row(j, carry):
        for cc in range(CTX_DIM // LANES):
            x = rows_v[j, pl.ds(cc * LANES, LANES)]
            slab_a[pl.ds(j * CTX_DIM + cc * LANES, LANES)] = x
            slab_b[pl.ds(j * CTX_DIM + cc * LANES, LANES)] = x
        return carry
    lax.fori_loop(0, SEQ, _flatten_row, 0)

    w0 = (n + 1) * CTX_DIM

    def _out_dst(v):
        return out_hbm.at[pl.ds((b * N_VERB + v) * SLAB, SLAB)]

    pltpu.sync_copy(ctxf_hbm.at[pl.ds(v0 * CTXW, CTXW)],
                    slab_a.at[pl.ds(w0, CTXW)])
    pltpu.async_copy(slab_a, _out_dst(v0), sem_a)
    pltpu.sync_copy(ctxf_hbm.at[pl.ds((v0 + 1) * CTXW, CTXW)],
                    slab_b.at[pl.ds(w0, CTXW)])
    pltpu.async_copy(slab_b, _out_dst(v0 + 1), sem_b)

    def _step(g, carry):
        v = v0 + 2 * g
        pltpu.make_async_copy(slab_a, _out_dst(v), sem_a).wait()
        pltpu.sync_copy(ctxf_hbm.at[pl.ds(v * CTXW, CTXW)],
                        slab_a.at[pl.ds(w0, CTXW)])
        pltpu.async_copy(slab_a, _out_dst(v), sem_a)
        pltpu.make_async_copy(slab_b, _out_dst(v + 1), sem_b).wait()
        pltpu.sync_copy(ctxf_hbm.at[pl.ds((v + 1) * CTXW, CTXW)],
                        slab_b.at[pl.ds(w0, CTXW)])
        pltpu.async_copy(slab_b, _out_dst(v + 1), sem_b)
        return carry

    lax.fori_loop(1, V_PER_W // 2, _step, 0)

    v_last = v0 + V_PER_W - 2
    pltpu.make_async_copy(slab_a, _out_dst(v_last), sem_a).wait()
    pltpu.make_async_copy(slab_b, _out_dst(v_last + 1), sem_b).wait()


def _sc_stage(tok_pad, nn, prefix, table, ctx_flat):
    mesh = plsc.VectorSubcoreMesh(core_axis_name="c", subcore_axis_name="s",
                                  num_cores=2, num_subcores=16)
    sc_fn = pl.kernel(
        _sc_body,
        out_type=(
            jax.ShapeDtypeStruct((B * N_VERB * SEQ * CTX_DIM,), jnp.float32),
            jax.ShapeDtypeStruct((B * PAD_SEQ,), jnp.int32),
        ),
        mesh=mesh,
        compiler_params=pltpu.CompilerParams(needs_layout_passes=False),
        scratch_types=[
            pltpu.VMEM((PAD_SEQ,), jnp.int32),
            pltpu.VMEM((16,), jnp.int32),
            pltpu.VMEM((N_CTX,), jnp.int32),
            pltpu.VMEM((G_ROWS,), jnp.int32),
            pltpu.VMEM((G_ROWS,), jnp.int32),
            pltpu.VMEM((G_ROWS, CTX_DIM), jnp.float32),
            pltpu.VMEM((SLAB,), jnp.float32),
            pltpu.VMEM((SLAB,), jnp.float32),
            pltpu.SemaphoreType.DMA,
            pltpu.SemaphoreType.DMA,
            pltpu.SemaphoreType.DMA,
        ],
    )
    return sc_fn(tok_pad, nn, prefix, table, ctx_flat)


def kernel_impl(nouns_token, nouns_numbers, ctx, token_embedding_weight,
                prompt_prefix_token):
    tok_pad = jnp.zeros((B, PAD_SEQ), jnp.int32).at[:, :SEQ].set(nouns_token)
    prefix = prompt_prefix_token.reshape(N_CTX).astype(jnp.int32)
    nn = nouns_numbers.astype(jnp.int32)

    out_flat, ct_flat = _sc_stage(tok_pad.reshape(B * PAD_SEQ), nn, prefix,
                                  token_embedding_weight,
                                  ctx.reshape(N_VERB * N_CTX * CTX_DIM))
    prompts = out_flat.reshape(B, N_VERB, SEQ, CTX_DIM)
    return prompts, ct_flat.reshape(B, PAD_SEQ)[:, :SEQ]


_jitted = None


def kernel(nouns_token, nouns_numbers, ctx, token_embedding_weight,
           prompt_prefix_token):
    global _jitted
    if _jitted is None:
        dev_sharding = jax.sharding.SingleDeviceSharding(jax.devices()[0])
        linear4d = jex_layout.Format(
            jex_layout.Layout(major_to_minor=(0, 1, 2, 3), tiling=()),
            dev_sharding)
        kernel_impl.__name__ = "kernel"
        _jitted = jax.jit(
            kernel_impl,
            out_shardings=(linear4d, jex_layout.Format(None, dev_sharding)))
    return _jitted(nouns_token, nouns_numbers, ctx, token_embedding_weight,
                   prompt_prefix_token)


# speedup vs baseline: 1.7940x; 1.7940x over previous
import jax
import jax.numpy as jnp
from jax import lax
from jax.experimental import pallas as pl
from jax.experimental.pallas import tpu as pltpu
import jax.experimental.pallas.tpu_sc as plsc

B = 16
SEQ = 77
N_CTX = 16
N_VERB = 64
CTX_DIM = 512

PAD_SEQ = 96
ROWS_PER_W = 48
V_BLK = 16
NBUF = 4


def _sc_gather_body(tok_hbm, nn_hbm, pre_hbm, table_hbm,
                    e_hbm, ct_hbm,
                    tok_v, nn_v, pre_v, idx_v, ct_v, rows_v, sem):
    c = lax.axis_index("c")
    s = lax.axis_index("s")
    wid = s * 2 + c
    b = wid // 2
    j0 = (wid % 2) * ROWS_PER_W

    pltpu.sync_copy(tok_hbm.at[pl.ds(b * PAD_SEQ, PAD_SEQ)], tok_v)
    pltpu.sync_copy(nn_hbm, nn_v)
    pltpu.sync_copy(pre_hbm, pre_v)

    bvec = jnp.full((16,), b, jnp.int32)
    n1 = plsc.load_gather(nn_v, [bvec])
    iota = lax.iota(jnp.int32, 16)

    for ci in range(ROWS_PER_W // 16):
        j = j0 + 16 * ci + iota
        in_ctx = (j > n1) & (j <= n1 + N_CTX)
        tidx = jnp.where(j <= n1, j, j - N_CTX)
        tidx = jnp.clip(tidx, 0, SEQ - 1)
        tok = plsc.load_gather(tok_v, [tidx])
        cidx = jnp.clip(j - 1 - n1, 0, N_CTX - 1)
        pre = plsc.load_gather(pre_v, [cidx])
        ct_v[pl.ds(16 * ci, 16)] = jnp.where(in_ctx, pre, tok)
        idx_v[pl.ds(16 * ci, 16)] = tok

    pltpu.async_copy(table_hbm.at[idx_v], rows_v, sem).wait()
    pltpu.sync_copy(rows_v, e_hbm.at[b, pl.ds(j0, ROWS_PER_W)])
    pltpu.sync_copy(ct_v, ct_hbm.at[pl.ds(b * PAD_SEQ + j0, ROWS_PER_W)])


def _tc_splice_body(nn_smem, e_ref, ctx_ref, out_hbm,
                    buf0, buf1, buf2, buf3, sem0, sem1, sem2, sem3):
    b = pl.program_id(0)
    n = nn_smem[b]
    bufs = [buf0, buf1, buf2, buf3]
    sems = [sem0, sem1, sem2, sem3]

    e = e_ref[0, :SEQ, :]
    for k in range(NBUF):
        buf, sem = bufs[k], sems[k]
        dst = out_hbm.at[b, pl.ds(k * V_BLK, V_BLK)]

        @pl.when(b > 0)
        def _():
            pltpu.make_async_copy(buf, out_hbm.at[b - 1, pl.ds(k * V_BLK,
                                                               V_BLK)],
                                  sem).wait()

        buf[:, :, :] = jnp.broadcast_to(e[None], (V_BLK, SEQ, CTX_DIM))
        ctx_blk = ctx_ref[pl.ds(k * V_BLK, V_BLK)]
        for nv in range(8):
            @pl.when(n == nv)
            def _():
                buf[:, nv + 1:nv + 1 + N_CTX, :] = ctx_blk
        pltpu.async_copy(buf, dst, sem)

    @pl.when(b == B - 1)
    def _():
        for k in range(NBUF):
            pltpu.make_async_copy(bufs[k],
                                  out_hbm.at[b, pl.ds(k * V_BLK, V_BLK)],
                                  sems[k]).wait()


def _sc_stage(tok_pad, nn, prefix, table):
    mesh = plsc.VectorSubcoreMesh(core_axis_name="c", subcore_axis_name="s",
                                  num_cores=2, num_subcores=16)
    sc_fn = pl.kernel(
        _sc_gather_body,
        out_type=(
            jax.ShapeDtypeStruct((B, PAD_SEQ, CTX_DIM), jnp.float32),
            jax.ShapeDtypeStruct((B * PAD_SEQ,), jnp.int32),
        ),
        mesh=mesh,
        compiler_params=pltpu.CompilerParams(needs_layout_passes=False),
        scratch_types=[
            pltpu.VMEM((PAD_SEQ,), jnp.int32),
            pltpu.VMEM((16,), jnp.int32),
            pltpu.VMEM((N_CTX,), jnp.int32),
            pltpu.VMEM((ROWS_PER_W,), jnp.int32),
            pltpu.VMEM((ROWS_PER_W,), jnp.int32),
            pltpu.VMEM((ROWS_PER_W, CTX_DIM), jnp.float32),
            pltpu.SemaphoreType.DMA,
        ],
    )
    return sc_fn(tok_pad, nn, prefix, table)


def _tc_splice(nn, e_pad, ctx):
    return pl.pallas_call(
        _tc_splice_body,
        grid=(B,),
        in_specs=[
            pl.BlockSpec(memory_space=pltpu.SMEM),
            pl.BlockSpec((1, PAD_SEQ, CTX_DIM), lambda b: (b, 0, 0)),
            pl.BlockSpec((N_VERB, N_CTX, CTX_DIM), lambda b: (0, 0, 0)),
        ],
        out_specs=pl.BlockSpec(memory_space=pltpu.MemorySpace.HBM),
        out_shape=jax.ShapeDtypeStruct((B, N_VERB, SEQ, CTX_DIM), jnp.float32),
        scratch_shapes=[pltpu.VMEM((V_BLK, SEQ, CTX_DIM), jnp.float32)
                        for _ in range(NBUF)] +
                       [pltpu.SemaphoreType.DMA for _ in range(NBUF)],
        compiler_params=pltpu.CompilerParams(
            dimension_semantics=("arbitrary",)),
    )(nn, e_pad, ctx)


@jax.jit
def kernel(nouns_token, nouns_numbers, ctx, token_embedding_weight,
           prompt_prefix_token):
    tok_pad = jnp.zeros((B, PAD_SEQ), jnp.int32).at[:, :SEQ].set(nouns_token)
    prefix = prompt_prefix_token.reshape(N_CTX).astype(jnp.int32)
    nn = nouns_numbers.astype(jnp.int32)

    e_pad, ct_flat = _sc_stage(tok_pad.reshape(B * PAD_SEQ), nn, prefix,
                               token_embedding_weight)
    prompts = _tc_splice(nn, e_pad, ctx)
    return prompts, ct_flat.reshape(B, PAD_SEQ)[:, :SEQ]
